# Initial kernel scaffold; baseline (speedup 1.0000x reference)
import jax
import jax.numpy as jnp
from jax.experimental import pallas as pl


def kernel(x):
    raise NotImplementedError("write your pallas kernel here")


# speedup vs baseline: 2221.2186x; 2221.2186x over previous
import functools
import math

import jax
import jax.numpy as jnp
import numpy as np
from jax.experimental import pallas as pl
from jax.experimental.pallas import tpu as pltpu

_N_SPIXELS = 256
_N_ITERS = 5


def _cells_layout(h, w, n_spixels):
    nw = int(math.sqrt(n_spixels * w / h) + 0.5)
    nh = int(math.sqrt(n_spixels * h / w) + 0.5)
    cw = int(math.ceil(w / nw))
    ch = int(math.ceil(h / nh))
    return nh, nw, ch, cw


def _consts(h, w, nh, nw, ch, cw, b, c):
    j_of_l = np.minimum(np.arange(w) // cw, nw - 1)
    E = np.zeros((3, w, nw), np.float32)
    colok = np.zeros((3, 1, w), np.float32)
    for t, dx in enumerate((-1, 0, 1)):
        jj = j_of_l + dx
        ok = (jj >= 0) & (jj < nw)
        jc = np.clip(jj, 0, nw - 1)
        E[t, np.arange(w), jc] = 1.0
        colok[t, 0] = ok.astype(np.float32)
    G = np.ascontiguousarray(np.transpose(E, (0, 2, 1)))
    R = np.zeros((b, b * c), np.float32)
    for bi in range(b):
        R[bi, bi * c:(bi + 1) * c] = 1.0
    return jnp.asarray(E), jnp.asarray(G), jnp.asarray(colok), jnp.asarray(R)


def _ssn_body(x_ref, e_ref, g_ref, colok_ref, rrep_ref, q_ref, spfp_ref,
              spf_s, num_s, den_s, *, nh, nw, ch, b, c, n_iters):
    i = pl.program_id(0)
    r = pl.program_id(1)
    w = x_ref.shape[-1]
    bc = b * c
    X = x_ref[...]
    X20 = X.reshape(bc, ch, w)

    @pl.when(jnp.logical_and(i == 0, r == 0))
    def _zero():
        num_s[...] = jnp.zeros_like(num_s)
        den_s[...] = jnp.zeros_like(den_s)

    @pl.when(i == 0)
    def _init():
        colsum = jnp.sum(X20, axis=1)
        contrib = jax.lax.dot_general(
            e_ref[1], colsum, (((0,), (1,)), ((), ())),
            preferred_element_type=jnp.float32)
        num_s[pl.ds(r * nw, nw), :] += contrib
        cnt = jax.lax.dot_general(
            e_ref[1], jnp.ones((b, w), jnp.float32), (((0,), (1,)), ((), ())),
            preferred_element_type=jnp.float32) * float(ch)
        den_s[pl.ds(r * nw, nw), :] += cnt

    @pl.when(i > 0)
    def _iterate():
        nd = []
        for dy in (-1, 0, 1):
            rn = r + dy
            row_ok = jnp.logical_and(rn >= 0, rn < nh)
            rp = jnp.clip(rn, 0, nh - 1)
            S = spf_s[pl.ds(rp * nw, nw), :]
            for t_dx in range(3):
                Map = jax.lax.dot_general(
                    S, g_ref[t_dx], (((0,), (0,)), ((), ())),
                    preferred_element_type=jnp.float32)
                diff = X - Map.reshape(b, c, 1, w)
                d = jnp.sum(diff * diff, axis=1)
                mask = jnp.logical_and(row_ok, colok_ref[t_dx] > 0.0)
                nd.append(jnp.where(mask, -d, jnp.float32(-1e16)))
        nds = jnp.stack(nd, axis=1)
        m = jnp.max(nds, axis=1, keepdims=True)
        ex = jnp.exp(nds - m)
        Q = ex / jnp.sum(ex, axis=1, keepdims=True)
        q_ref[...] = Q
        for t_dy, dy in enumerate((-1, 0, 1)):
            rp = jnp.clip(r + dy, 0, nh - 1)
            cn = jnp.zeros((nw, bc), jnp.float32)
            cd = jnp.zeros((nw, b), jnp.float32)
            for t_dx in range(3):
                k = t_dy * 3 + t_dx
                qk = Q[:, k]
                qcol = jnp.sum(qk, axis=1)
                wc = jnp.sum(qk[:, None] * X, axis=2)
                cn = cn + jax.lax.dot_general(
                    e_ref[t_dx], wc.reshape(bc, w), (((0,), (1,)), ((), ())),
                    preferred_element_type=jnp.float32)
                cd = cd + jax.lax.dot_general(
                    e_ref[t_dx], qcol, (((0,), (1,)), ((), ())),
                    preferred_element_type=jnp.float32)
            num_s[pl.ds(rp * nw, nw), :] += cn
            den_s[pl.ds(rp * nw, nw), :] += cd

    @pl.when(r == nh - 1)
    def _finalize():
        den = den_s[...]
        den_bc = jax.lax.dot_general(
            den, rrep_ref[...], (((1,), (0,)), ((), ())),
            preferred_element_type=jnp.float32)
        denom = jnp.where(i == 0, jnp.maximum(den_bc, 1.0), den_bc + 1e-16)
        spf = num_s[...] / denom
        spf_s[...] = spf
        num_s[...] = jnp.zeros_like(num_s)
        den_s[...] = jnp.zeros_like(den_s)

        @pl.when(i == n_iters)
        def _emit_spf():
            spfp_ref[...] = spf


@jax.jit
def kernel(x):
    b, c, h, w = x.shape
    nh, nw, ch, cw = _cells_layout(h, w, _N_SPIXELS)
    assert nh * ch == h and nw * cw == w, "kernel assumes even cell tiling"
    n_sp = nh * nw
    E, G, colok, R = _consts(h, w, nh, nw, ch, cw, b, c)
    grid = (_N_ITERS + 1, nh)
    body = functools.partial(_ssn_body, nh=nh, nw=nw, ch=ch, b=b, c=c,
                             n_iters=_N_ITERS)
    q, spf_p = pl.pallas_call(
        body,
        grid=grid,
        in_specs=[
            pl.BlockSpec((b, c, ch, w), lambda i, r: (0, 0, r, 0)),
            pl.BlockSpec((3, w, nw), lambda i, r: (0, 0, 0)),
            pl.BlockSpec((3, nw, w), lambda i, r: (0, 0, 0)),
            pl.BlockSpec((3, 1, w), lambda i, r: (0, 0, 0)),
            pl.BlockSpec((b, b * c), lambda i, r: (0, 0)),
        ],
        out_specs=[
            pl.BlockSpec((b, 9, ch, w), lambda i, r: (0, 0, r, 0)),
            pl.BlockSpec((n_sp, b * c), lambda i, r: (0, 0)),
        ],
        out_shape=[
            jax.ShapeDtypeStruct((b, 9, h, w), jnp.float32),
            jax.ShapeDtypeStruct((n_sp, b * c), jnp.float32),
        ],
        scratch_shapes=[
            pltpu.VMEM((n_sp, b * c), jnp.float32),
            pltpu.VMEM((n_sp, b * c), jnp.float32),
            pltpu.VMEM((n_sp, b), jnp.float32),
        ],
        compiler_params=pltpu.CompilerParams(
            dimension_semantics=("arbitrary", "arbitrary")),
    )(x, E, G, colok, R)
    spf_out = spf_p.T.reshape(b, c, n_sp)
    return (q, x, spf_out, x)
